# baseline (speedup 1.0000x reference)
import functools

import jax
import jax.numpy as jnp
from jax import lax
from jax.experimental import pallas as pl
from jax.experimental.pallas import tpu as pltpu
from jax.experimental.pallas import tpu_sc as plsc

N = 10000
E = 320000
D = 128
G = 64

NC = 2
NS = 16
NW = NC * NS
BLK = 128
NB = 2
EB_PER_TILE = 80
E_PAD = NW * BLK * EB_PER_TILE
NBLK = E_PAD // BLK
CHUNK = 16
NCHUNK = EB_PER_TILE // CHUNK
NGC = CHUNK // NB
DEG_K = 8
N_PAD = 10112
ROWS_PER_TILE = N_PAD // NS
DEGW = 16

@functools.cache
def _sc_kernels():
    mesh = plsc.VectorSubcoreMesh(core_axis_name="c", subcore_axis_name="s",
                                  num_cores=NC, num_subcores=NS)
    deg = functools.partial(
        pl.kernel,
        out_type=jax.ShapeDtypeStruct((NC, N_PAD, D), jnp.float32),
        mesh=mesh,
        scratch_types=[
            pltpu.VMEM_SHARED((N_PAD, D), jnp.float32),
            pltpu.VMEM((BLK, D), jnp.float32),
            pltpu.VMEM((EB_PER_TILE, BLK), jnp.int32),
            pltpu.SemaphoreType.DMA,
        ],
    )(_sc_deg_body)
    scatter = functools.partial(
        pl.kernel,
        out_type=jax.ShapeDtypeStruct((NC, N_PAD, D), jnp.float32),
        mesh=mesh,
        scratch_types=[
            pltpu.VMEM_SHARED((N_PAD, D), jnp.float32),
            pltpu.VMEM((NB, BLK, D), jnp.float32),
            pltpu.VMEM((2 * CHUNK, BLK), jnp.int32),
            pltpu.VMEM((2 * CHUNK, BLK), jnp.int32),
            [pltpu.SemaphoreType.DMA] * NB,
            [pltpu.SemaphoreType.DMA] * NB,
            pltpu.SemaphoreType.DMA,
        ],
    )(_sc_scatter_body)
    return deg, scatter


def _sc_deg_body(dst_hbm, ones_hbm, zeros_hbm, out_hbm, deg_sh, ones_v,
                 dst_all, sem):
    c = lax.axis_index("c")
    s = lax.axis_index("s")
    wid = s * NC + c
    base = wid * EB_PER_TILE

    pltpu.sync_copy(ones_hbm, ones_v)
    pltpu.sync_copy(dst_hbm.at[pl.ds(base, EB_PER_TILE)], dst_all)
    pltpu.sync_copy(zeros_hbm,
                    deg_sh.at[pl.ds(s * ROWS_PER_TILE, ROWS_PER_TILE)])
    plsc.subcore_barrier()

    def _group(g, _):
        descs = []
        for k in range(DEG_K):
            descs.append(pltpu.async_copy(
                ones_v, deg_sh.at[dst_all.at[g * DEG_K + k]], sem, add=True))
        for dsc in descs:
            dsc.wait()
        return 0

    lax.fori_loop(0, EB_PER_TILE // DEG_K, _group, 0)
    plsc.subcore_barrier()
    pltpu.sync_copy(
        deg_sh.at[pl.ds(s * ROWS_PER_TILE, ROWS_PER_TILE)],
        out_hbm.at[c, pl.ds(s * ROWS_PER_TILE, ROWS_PER_TILE)],
    )


def _sc_scatter_body(y_hbm, src_hbm, dst_hbm, out_hbm, acc_sh, rows_v,
                     src_ch, dst_ch, gsems, ssems, isem):
    c = lax.axis_index("c")
    s = lax.axis_index("s")
    wid = s * NC + c
    base = wid * EB_PER_TILE

    pltpu.sync_copy(src_hbm.at[pl.ds(base, CHUNK)], src_ch.at[pl.ds(0, CHUNK)])
    pltpu.sync_copy(dst_hbm.at[pl.ds(base, CHUNK)], dst_ch.at[pl.ds(0, CHUNK)])
    pltpu.sync_copy(
        y_hbm.at[pl.ds(s * ROWS_PER_TILE, ROWS_PER_TILE)],
        acc_sh.at[pl.ds(s * ROWS_PER_TILE, ROWS_PER_TILE)],
    )
    plsc.subcore_barrier()

    def _gather(row, b):
        return pltpu.async_copy(y_hbm.at[src_ch.at[row]], rows_v.at[b],
                                gsems[b])

    def _scatter(row, b):
        return pltpu.async_copy(rows_v.at[b], acc_sh.at[dst_ch.at[row]],
                                ssems[b], add=True)

    for k in range(NCHUNK):
        par = (k & 1) * CHUNK
        nxt = ((k + 1) & 1) * CHUNK
        idx_pf = []
        if k + 1 < NCHUNK:
            off = base + (k + 1) * CHUNK
            idx_pf.append(pltpu.async_copy(
                src_hbm.at[pl.ds(off, CHUNK)],
                src_ch.at[pl.ds(nxt, CHUNK)], isem))
            idx_pf.append(pltpu.async_copy(
                dst_hbm.at[pl.ds(off, CHUNK)],
                dst_ch.at[pl.ds(nxt, CHUNK)], isem))

        for b in range(NB):
            _gather(par + b, b)

        def _steady(g, _, par=par):
            o = par + g * NB
            for b in range(NB):
                pltpu.make_async_copy(y_hbm.at[src_ch.at[o + b]],
                                      rows_v.at[b], gsems[b]).wait()
                _scatter(o + b, b)
            for b in range(NB):
                pltpu.make_async_copy(rows_v.at[b],
                                      acc_sh.at[dst_ch.at[o + b]],
                                      ssems[b]).wait()
                _gather(o + NB + b, b)
            return 0

        lax.fori_loop(0, NGC - 1, _steady, 0)

        o = par + (NGC - 1) * NB
        for b in range(NB):
            pltpu.make_async_copy(y_hbm.at[src_ch.at[o + b]], rows_v.at[b],
                                  gsems[b]).wait()
            _scatter(o + b, b)
        for b in range(NB):
            pltpu.make_async_copy(rows_v.at[b], acc_sh.at[dst_ch.at[o + b]],
                                  ssems[b]).wait()
        for dsc in idx_pf:
            dsc.wait()

    plsc.subcore_barrier()
    pltpu.sync_copy(
        acc_sh.at[pl.ds(s * ROWS_PER_TILE, ROWS_PER_TILE)],
        out_hbm.at[c, pl.ds(s * ROWS_PER_TILE, ROWS_PER_TILE)],
    )


def _tc_stage1_body(x_ref, w_ref, degp_ref, y_ref, d_ref):
    deg = degp_ref[0, :, 0:1] + degp_ref[1, :, 0:1] + 1.0
    rows = lax.broadcasted_iota(jnp.int32, (N_PAD, 1), 0)
    d = jnp.where(rows < N, lax.rsqrt(deg), 0.0)
    d_ref[...] = d
    y_ref[...] = jnp.dot(x_ref[...], w_ref[...],
                         preferred_element_type=jnp.float32) * d


def _tc_stage2_body(sp_ref, y1_ref, d_ref, b1_ref, w2_ref, y2_ref):
    d = d_ref[...]
    agg = sp_ref[0] + sp_ref[1] - y1_ref[...]
    h = jnp.maximum(d * agg + b1_ref[...], 0.0)
    y2_ref[...] = jnp.dot(h, w2_ref[...],
                          preferred_element_type=jnp.float32) * d


def _tc_stage3_body(sp_ref, y2_ref, d_ref, b2_ref, batch_ref, out_ref):
    z = d_ref[...] * (sp_ref[0] + sp_ref[1] - y2_ref[...])
    gid = lax.broadcasted_iota(jnp.int32, (G, N_PAD), 0)
    oh = (gid == batch_ref[...]).astype(jnp.float32)
    pooled = jnp.dot(oh, z, preferred_element_type=jnp.float32)
    counts = jnp.sum(oh, axis=1, keepdims=True)
    out_ref[...] = (pooled + counts * b2_ref[...]) / jnp.maximum(counts, 1.0)


_tc_stage1 = pl.pallas_call(
    _tc_stage1_body,
    out_shape=(
        jax.ShapeDtypeStruct((N_PAD, D), jnp.float32),
        jax.ShapeDtypeStruct((N_PAD, 1), jnp.float32),
    ),
)

_tc_stage2 = pl.pallas_call(
    _tc_stage2_body,
    out_shape=jax.ShapeDtypeStruct((N_PAD, D), jnp.float32),
)

_tc_stage3 = pl.pallas_call(
    _tc_stage3_body,
    out_shape=jax.ShapeDtypeStruct((G, D), jnp.float32),
)


def kernel(x, edge_index, batch, W1, b1, W2, b2):
    src = edge_index[0].astype(jnp.int32)
    dst = edge_index[1].astype(jnp.int32)
    pad = jnp.full((E_PAD - E,), N, jnp.int32)
    src_p = jnp.concatenate([src, pad]).reshape(NBLK, BLK)
    dst_p = jnp.concatenate([dst, pad]).reshape(NBLK, BLK)
    x_p = jnp.pad(x, ((0, N_PAD - N), (0, 0)))
    batch_p = jnp.pad(batch.astype(jnp.int32), (0, N_PAD - N),
                      constant_values=G).reshape(1, N_PAD)
    b1r = b1.reshape(1, D)
    b2r = b2.reshape(1, D)

    sc_deg, sc_scatter = _sc_kernels()
    ones_c = jnp.ones((BLK, D), jnp.float32)
    zeros_c = jnp.zeros((ROWS_PER_TILE, D), jnp.float32)
    degp = sc_deg(dst_p, ones_c, zeros_c)
    y1, d = _tc_stage1(x_p, W1, degp)
    s1 = sc_scatter(y1, src_p, dst_p)
    y2 = _tc_stage2(s1, y1, d, b1r, W2)
    s2 = sc_scatter(y2, src_p, dst_p)
    return _tc_stage3(s2, y2, d, b2r, batch_p)


# speedup vs baseline: 1.9949x; 1.9949x over previous
import functools

import jax
import jax.numpy as jnp
from jax import lax
from jax.experimental import pallas as pl
from jax.experimental.pallas import tpu as pltpu
from jax.experimental.pallas import tpu_sc as plsc

N = 10000
E = 320000
D = 128
G = 64

NC = 2
NS = 16
NW = NC * NS
BLK = 128
DH = D // NC
E_PAD = NW * BLK * 80
NBLK = E_PAD // BLK
EB_DEG = NBLK // NW
EBT = NBLK // NS
NB = 4
CHUNK = 16
NCHUNK = EBT // CHUNK
NGC = CHUNK // NB
DEG_K = 8
N_PAD = 10112
ROWS_PER_TILE = N_PAD // NS

@functools.cache
def _sc_kernels():
    mesh = plsc.VectorSubcoreMesh(core_axis_name="c", subcore_axis_name="s",
                                  num_cores=NC, num_subcores=NS)
    deg = functools.partial(
        pl.kernel,
        out_type=jax.ShapeDtypeStruct((NC, N_PAD, D), jnp.float32),
        mesh=mesh,
        scratch_types=[
            pltpu.VMEM_SHARED((N_PAD, D), jnp.float32),
            pltpu.VMEM((BLK, D), jnp.float32),
            pltpu.VMEM((EB_DEG, BLK), jnp.int32),
            pltpu.SemaphoreType.DMA,
        ],
    )(_sc_deg_body)
    scatter = functools.partial(
        pl.kernel,
        out_type=jax.ShapeDtypeStruct((NC, N_PAD, DH), jnp.float32),
        mesh=mesh,
        scratch_types=[
            pltpu.VMEM_SHARED((N_PAD, DH), jnp.float32),
            pltpu.VMEM_SHARED((N_PAD, DH), jnp.float32),
            pltpu.VMEM((NB, BLK, DH), jnp.float32),
            pltpu.VMEM((2 * CHUNK, BLK), jnp.int32),
            pltpu.VMEM((2 * CHUNK, BLK), jnp.int32),
            [pltpu.SemaphoreType.DMA] * NB,
            [pltpu.SemaphoreType.DMA] * NB,
            pltpu.SemaphoreType.DMA,
        ],
        compiler_params=pltpu.CompilerParams(use_tc_tiling_on_sc=False),
    )(_sc_scatter_body)
    return deg, scatter


def _sc_deg_body(dst_hbm, ones_hbm, zeros_hbm, out_hbm, deg_sh, ones_v,
                 dst_all, sem):
    c = lax.axis_index("c")
    s = lax.axis_index("s")
    wid = s * NC + c
    base = wid * EB_DEG

    pltpu.sync_copy(ones_hbm, ones_v)
    pltpu.sync_copy(dst_hbm.at[pl.ds(base, EB_DEG)], dst_all)
    pltpu.sync_copy(zeros_hbm,
                    deg_sh.at[pl.ds(s * ROWS_PER_TILE, ROWS_PER_TILE)])
    plsc.subcore_barrier()

    def _group(g, _):
        descs = []
        for k in range(DEG_K):
            descs.append(pltpu.async_copy(
                ones_v, deg_sh.at[dst_all.at[g * DEG_K + k]], sem, add=True))
        for dsc in descs:
            dsc.wait()
        return 0

    lax.fori_loop(0, EB_DEG // DEG_K, _group, 0)
    plsc.subcore_barrier()
    pltpu.sync_copy(
        deg_sh.at[pl.ds(s * ROWS_PER_TILE, ROWS_PER_TILE)],
        out_hbm.at[c, pl.ds(s * ROWS_PER_TILE, ROWS_PER_TILE)],
    )


def _sc_scatter_body(yh_hbm, src_hbm, dst_hbm, out_hbm, ysh, acc_sh, rows_v,
                     src_ch, dst_ch, gsems, ssems, isem):
    c = lax.axis_index("c")
    s = lax.axis_index("s")
    base = s * EBT

    pltpu.sync_copy(yh_hbm.at[c, pl.ds(s * ROWS_PER_TILE, ROWS_PER_TILE)],
                    ysh.at[pl.ds(s * ROWS_PER_TILE, ROWS_PER_TILE)])
    pltpu.sync_copy(yh_hbm.at[c, pl.ds(s * ROWS_PER_TILE, ROWS_PER_TILE)],
                    acc_sh.at[pl.ds(s * ROWS_PER_TILE, ROWS_PER_TILE)])
    pltpu.sync_copy(src_hbm.at[pl.ds(base, CHUNK)], src_ch.at[pl.ds(0, CHUNK)])
    pltpu.sync_copy(dst_hbm.at[pl.ds(base, CHUNK)], dst_ch.at[pl.ds(0, CHUNK)])
    plsc.subcore_barrier()

    def _gather(row, b):
        return pltpu.async_copy(ysh.at[src_ch.at[row]], rows_v.at[b],
                                gsems[b])

    def _scatter(row, b):
        return pltpu.async_copy(rows_v.at[b], acc_sh.at[dst_ch.at[row]],
                                ssems[b], add=True)

    for k in range(NCHUNK):
        par = (k & 1) * CHUNK
        nxt = ((k + 1) & 1) * CHUNK
        idx_pf = []
        if k + 1 < NCHUNK:
            off = base + (k + 1) * CHUNK
            idx_pf.append(pltpu.async_copy(
                src_hbm.at[pl.ds(off, CHUNK)],
                src_ch.at[pl.ds(nxt, CHUNK)], isem))
            idx_pf.append(pltpu.async_copy(
                dst_hbm.at[pl.ds(off, CHUNK)],
                dst_ch.at[pl.ds(nxt, CHUNK)], isem))

        for b in range(NB):
            _gather(par + b, b)

        def _steady(g, _, par=par):
            o = par + g * NB
            for b in range(NB):
                pltpu.make_async_copy(ysh.at[src_ch.at[o + b]],
                                      rows_v.at[b], gsems[b]).wait()
                _scatter(o + b, b)
            for b in range(NB):
                pltpu.make_async_copy(rows_v.at[b],
                                      acc_sh.at[dst_ch.at[o + b]],
                                      ssems[b]).wait()
                _gather(o + NB + b, b)
            return 0

        lax.fori_loop(0, NGC - 1, _steady, 0)

        o = par + (NGC - 1) * NB
        for b in range(NB):
            pltpu.make_async_copy(ysh.at[src_ch.at[o + b]], rows_v.at[b],
                                  gsems[b]).wait()
            _scatter(o + b, b)
        for b in range(NB):
            pltpu.make_async_copy(rows_v.at[b], acc_sh.at[dst_ch.at[o + b]],
                                  ssems[b]).wait()
        for dsc in idx_pf:
            dsc.wait()

    plsc.subcore_barrier()
    pltpu.sync_copy(
        acc_sh.at[pl.ds(s * ROWS_PER_TILE, ROWS_PER_TILE)],
        out_hbm.at[c, pl.ds(s * ROWS_PER_TILE, ROWS_PER_TILE)],
    )


def _tc_stage1_body(x_ref, w_ref, degp_ref, y_ref, d_ref):
    deg = degp_ref[0, :, 0:1] + degp_ref[1, :, 0:1] + 1.0
    rows = lax.broadcasted_iota(jnp.int32, (N_PAD, 1), 0)
    d = jnp.where(rows < N, lax.rsqrt(deg), 0.0)
    d_ref[...] = d
    y = jnp.dot(x_ref[...], w_ref[...],
                preferred_element_type=jnp.float32) * d
    y_ref[0] = y[:, :DH]
    y_ref[1] = y[:, DH:]


def _tc_stage2_body(sp_ref, y1_ref, d_ref, b1_ref, w2_ref, y2_ref):
    d = d_ref[...]
    agg = jnp.concatenate([sp_ref[0] - y1_ref[0], sp_ref[1] - y1_ref[1]],
                          axis=1)
    h = jnp.maximum(d * agg + b1_ref[...], 0.0)
    y2 = jnp.dot(h, w2_ref[...], preferred_element_type=jnp.float32) * d
    y2_ref[0] = y2[:, :DH]
    y2_ref[1] = y2[:, DH:]


def _tc_stage3_body(sp_ref, y2_ref, d_ref, b2_ref, batch_ref, out_ref):
    z = d_ref[...] * jnp.concatenate(
        [sp_ref[0] - y2_ref[0], sp_ref[1] - y2_ref[1]], axis=1)
    gid = lax.broadcasted_iota(jnp.int32, (G, N_PAD), 0)
    oh = (gid == batch_ref[...]).astype(jnp.float32)
    pooled = jnp.dot(oh, z, preferred_element_type=jnp.float32)
    counts = jnp.sum(oh, axis=1, keepdims=True)
    out_ref[...] = (pooled + counts * b2_ref[...]) / jnp.maximum(counts, 1.0)


_halves = jax.ShapeDtypeStruct((NC, N_PAD, DH), jnp.float32)

_tc_stage1 = pl.pallas_call(
    _tc_stage1_body,
    out_shape=(_halves, jax.ShapeDtypeStruct((N_PAD, 1), jnp.float32)),
)

_tc_stage2 = pl.pallas_call(_tc_stage2_body, out_shape=_halves)

_tc_stage3 = pl.pallas_call(
    _tc_stage3_body,
    out_shape=jax.ShapeDtypeStruct((G, D), jnp.float32),
)


def kernel(x, edge_index, batch, W1, b1, W2, b2):
    src = edge_index[0].astype(jnp.int32)
    dst = edge_index[1].astype(jnp.int32)
    pad = jnp.full((E_PAD - E,), N, jnp.int32)
    src_p = jnp.concatenate([src, pad]).reshape(NBLK, BLK)
    dst_p = jnp.concatenate([dst, pad]).reshape(NBLK, BLK)
    x_p = jnp.pad(x, ((0, N_PAD - N), (0, 0)))
    batch_p = jnp.pad(batch.astype(jnp.int32), (0, N_PAD - N),
                      constant_values=G).reshape(1, N_PAD)
    b1r = b1.reshape(1, D)
    b2r = b2.reshape(1, D)

    sc_deg, sc_scatter = _sc_kernels()
    ones_c = jnp.ones((BLK, D), jnp.float32)
    zeros_c = jnp.zeros((ROWS_PER_TILE, D), jnp.float32)
    degp = sc_deg(dst_p, ones_c, zeros_c)
    y1h, d = _tc_stage1(x_p, W1, degp)
    s1 = sc_scatter(y1h, src_p, dst_p)
    y2h = _tc_stage2(s1, y1h, d, b1r, W2)
    s2 = sc_scatter(y2h, src_p, dst_p)
    return _tc_stage3(s2, y2h, d, b2r, batch_p)


# speedup vs baseline: 2.2309x; 1.1183x over previous
import functools

import jax
import jax.numpy as jnp
from jax import lax
from jax.experimental import pallas as pl
from jax.experimental.pallas import tpu as pltpu
from jax.experimental.pallas import tpu_sc as plsc

N = 10000
E = 320000
D = 128
G = 64

NC = 2
NS = 16
NW = NC * NS
BLK = 128
DH = D // NC
E_PAD = NW * BLK * 80
NBLK = E_PAD // BLK
EB_DEG = NBLK // NW
EBT = NBLK // NS
NB = 4
CHUNK = 32
NCHUNK = EBT // CHUNK
NGC = CHUNK // NB
DEG_K = 8
DEGW = 16
N_PAD = 10112
ROWS_PER_TILE = N_PAD // NS

@functools.cache
def _sc_kernels():
    mesh = plsc.VectorSubcoreMesh(core_axis_name="c", subcore_axis_name="s",
                                  num_cores=NC, num_subcores=NS)
    deg = functools.partial(
        pl.kernel,
        out_type=jax.ShapeDtypeStruct((NC, N_PAD, DEGW), jnp.float32),
        mesh=mesh,
        scratch_types=[
            pltpu.VMEM_SHARED((N_PAD, DEGW), jnp.float32),
            pltpu.VMEM((BLK, DEGW), jnp.float32),
            pltpu.VMEM((EB_DEG, BLK), jnp.int32),
            pltpu.SemaphoreType.DMA,
        ],
        compiler_params=pltpu.CompilerParams(use_tc_tiling_on_sc=False, skip_device_barrier=True),
    )(_sc_deg_body)
    scatter = functools.partial(
        pl.kernel,
        out_type=jax.ShapeDtypeStruct((NC, N_PAD, DH), jnp.float32),
        mesh=mesh,
        scratch_types=[
            pltpu.VMEM_SHARED((N_PAD, DH), jnp.float32),
            pltpu.VMEM_SHARED((N_PAD, DH), jnp.float32),
            pltpu.VMEM((NB, BLK, DH), jnp.float32),
            pltpu.VMEM((2 * CHUNK, BLK), jnp.int32),
            pltpu.VMEM((2 * CHUNK, BLK), jnp.int32),
            [pltpu.SemaphoreType.DMA] * NB,
            [pltpu.SemaphoreType.DMA] * NB,
            pltpu.SemaphoreType.DMA,
        ],
        compiler_params=pltpu.CompilerParams(use_tc_tiling_on_sc=False, skip_device_barrier=True),
    )(_sc_scatter_body)
    return deg, scatter


def _sc_deg_body(dst_hbm, ones_hbm, zeros_hbm, out_hbm, deg_sh, ones_v,
                 dst_all, sem):
    c = lax.axis_index("c")
    s = lax.axis_index("s")
    wid = s * NC + c
    base = wid * EB_DEG

    pltpu.sync_copy(ones_hbm, ones_v)
    pltpu.sync_copy(dst_hbm.at[pl.ds(base, EB_DEG)], dst_all)
    pltpu.sync_copy(zeros_hbm,
                    deg_sh.at[pl.ds(s * ROWS_PER_TILE, ROWS_PER_TILE)])
    plsc.subcore_barrier()

    def _group(g, _):
        descs = []
        for k in range(DEG_K):
            descs.append(pltpu.async_copy(
                ones_v, deg_sh.at[dst_all.at[g * DEG_K + k]], sem, add=True))
        for dsc in descs:
            dsc.wait()
        return 0

    lax.fori_loop(0, EB_DEG // DEG_K, _group, 0)
    plsc.subcore_barrier()
    pltpu.sync_copy(
        deg_sh.at[pl.ds(s * ROWS_PER_TILE, ROWS_PER_TILE)],
        out_hbm.at[c, pl.ds(s * ROWS_PER_TILE, ROWS_PER_TILE)],
    )


def _sc_scatter_body(yh_hbm, src_hbm, dst_hbm, out_hbm, ysh, acc_sh, rows_v,
                     src_ch, dst_ch, gsems, ssems, isem):
    c = lax.axis_index("c")
    s = lax.axis_index("s")
    base = s * EBT

    pltpu.sync_copy(yh_hbm.at[c, pl.ds(s * ROWS_PER_TILE, ROWS_PER_TILE)],
                    ysh.at[pl.ds(s * ROWS_PER_TILE, ROWS_PER_TILE)])
    pltpu.sync_copy(yh_hbm.at[c, pl.ds(s * ROWS_PER_TILE, ROWS_PER_TILE)],
                    acc_sh.at[pl.ds(s * ROWS_PER_TILE, ROWS_PER_TILE)])
    pltpu.sync_copy(src_hbm.at[pl.ds(base, CHUNK)], src_ch.at[pl.ds(0, CHUNK)])
    pltpu.sync_copy(dst_hbm.at[pl.ds(base, CHUNK)], dst_ch.at[pl.ds(0, CHUNK)])
    plsc.subcore_barrier()

    def _gather(row, b):
        return pltpu.async_copy(ysh.at[src_ch.at[row]], rows_v.at[b],
                                gsems[b])

    def _scatter(row, b):
        return pltpu.async_copy(rows_v.at[b], acc_sh.at[dst_ch.at[row]],
                                ssems[b], add=True)

    for k in range(NCHUNK):
        par = (k & 1) * CHUNK
        nxt = ((k + 1) & 1) * CHUNK
        idx_pf = []
        if k + 1 < NCHUNK:
            off = base + (k + 1) * CHUNK
            idx_pf.append(pltpu.async_copy(
                src_hbm.at[pl.ds(off, CHUNK)],
                src_ch.at[pl.ds(nxt, CHUNK)], isem))
            idx_pf.append(pltpu.async_copy(
                dst_hbm.at[pl.ds(off, CHUNK)],
                dst_ch.at[pl.ds(nxt, CHUNK)], isem))

        for b in range(NB):
            _gather(par + b, b)

        def _steady(g, _, par=par):
            o = par + g * NB
            for b in range(NB):
                pltpu.make_async_copy(ysh.at[src_ch.at[o + b]],
                                      rows_v.at[b], gsems[b]).wait()
                _scatter(o + b, b)
            for b in range(NB):
                pltpu.make_async_copy(rows_v.at[b],
                                      acc_sh.at[dst_ch.at[o + b]],
                                      ssems[b]).wait()
                _gather(o + NB + b, b)
            return 0

        lax.fori_loop(0, NGC - 1, _steady, 0)

        o = par + (NGC - 1) * NB
        for b in range(NB):
            pltpu.make_async_copy(ysh.at[src_ch.at[o + b]], rows_v.at[b],
                                  gsems[b]).wait()
            _scatter(o + b, b)
        for b in range(NB):
            pltpu.make_async_copy(rows_v.at[b], acc_sh.at[dst_ch.at[o + b]],
                                  ssems[b]).wait()
        for dsc in idx_pf:
            dsc.wait()

    plsc.subcore_barrier()
    pltpu.sync_copy(
        acc_sh.at[pl.ds(s * ROWS_PER_TILE, ROWS_PER_TILE)],
        out_hbm.at[c, pl.ds(s * ROWS_PER_TILE, ROWS_PER_TILE)],
    )


def _tc_stage1_body(x_ref, w_ref, degp_ref, y_ref, d_ref):
    deg = degp_ref[0, :, 0:1] + degp_ref[1, :, 0:1] + 1.0
    rows = lax.broadcasted_iota(jnp.int32, (N_PAD, 1), 0)
    d = jnp.where(rows < N, lax.rsqrt(deg), 0.0)
    d_ref[...] = d
    y = jnp.dot(x_ref[...], w_ref[...],
                preferred_element_type=jnp.float32) * d
    y_ref[0] = y[:, :DH]
    y_ref[1] = y[:, DH:]


def _tc_stage2_body(sp_ref, d_ref, b1_ref, w2_ref, y2_ref):
    d = d_ref[...]
    agg = jnp.concatenate([sp_ref[0], sp_ref[1]], axis=1)
    h = jnp.maximum(d * agg + b1_ref[...], 0.0)
    y2 = jnp.dot(h, w2_ref[...], preferred_element_type=jnp.float32) * d
    y2_ref[0] = y2[:, :DH]
    y2_ref[1] = y2[:, DH:]


def _tc_stage3_body(sp_ref, d_ref, b2_ref, batch_ref, out_ref):
    z = d_ref[...] * jnp.concatenate([sp_ref[0], sp_ref[1]], axis=1)
    gid = lax.broadcasted_iota(jnp.int32, (G, N_PAD), 0)
    oh = (gid == batch_ref[...]).astype(jnp.float32)
    pooled = jnp.dot(oh, z, preferred_element_type=jnp.float32)
    counts = jnp.sum(oh, axis=1, keepdims=True)
    out_ref[...] = (pooled + counts * b2_ref[...]) / jnp.maximum(counts, 1.0)


_halves = jax.ShapeDtypeStruct((NC, N_PAD, DH), jnp.float32)

_tc_stage1 = pl.pallas_call(
    _tc_stage1_body,
    out_shape=(_halves, jax.ShapeDtypeStruct((N_PAD, 1), jnp.float32)),
)

_tc_stage2 = pl.pallas_call(_tc_stage2_body, out_shape=_halves)

_tc_stage3 = pl.pallas_call(
    _tc_stage3_body,
    out_shape=jax.ShapeDtypeStruct((G, D), jnp.float32),
)


def kernel(x, edge_index, batch, W1, b1, W2, b2):
    src = edge_index[0].astype(jnp.int32)
    dst = edge_index[1].astype(jnp.int32)
    pad = jnp.full((E_PAD - E,), N, jnp.int32)
    src_p = jnp.concatenate([src, pad]).reshape(NBLK, BLK)
    dst_p = jnp.concatenate([dst, pad]).reshape(NBLK, BLK)
    x_p = jnp.pad(x, ((0, N_PAD - N), (0, 0)))
    batch_p = jnp.pad(batch.astype(jnp.int32), (0, N_PAD - N),
                      constant_values=G).reshape(1, N_PAD)
    b1r = b1.reshape(1, D)
    b2r = b2.reshape(1, D)

    sc_deg, sc_scatter = _sc_kernels()
    ones_c = jnp.ones((BLK, DEGW), jnp.float32)
    zeros_c = jnp.zeros((ROWS_PER_TILE, DEGW), jnp.float32)
    degp = sc_deg(dst_p, ones_c, zeros_c)
    y1h, d = _tc_stage1(x_p, W1, degp)
    s1 = sc_scatter(y1h, src_p, dst_p)
    y2h = _tc_stage2(s1, d, b1r, W2)
    s2 = sc_scatter(y2h, src_p, dst_p)
    return _tc_stage3(s2, d, b2r, batch_p)
